# baseline (speedup 1.0000x reference)
import jax
import jax.numpy as jnp
from jax.experimental import pallas as pl
from jax.experimental.pallas import tpu as pltpu


def _pe_kernel(row_ref, col_ref, out_ref):
    hb = row_ref.shape[0]
    w = col_ref.shape[0]
    c = col_ref.shape[1]
    ce_t = col_ref[...].T
    re_t = row_ref[...].T
    out_ref[0:c] = jnp.broadcast_to(ce_t[:, None, :], (c, hb, w))
    out_ref[c : 2 * c] = jnp.broadcast_to(re_t[:, :, None], (c, hb, w))


def kernel(x, row_embed, col_embed):
    h, w = x.shape[-2], x.shape[-1]
    c = row_embed.shape[1]
    nh = 4
    hb = h // nh
    out = pl.pallas_call(
        _pe_kernel,
        grid=(nh,),
        in_specs=[
            pl.BlockSpec((hb, c), lambda k: (k, 0)),
            pl.BlockSpec((w, c), lambda k: (0, 0)),
        ],
        out_specs=pl.BlockSpec((2 * c, hb, w), lambda k: (0, k, 0)),
        out_shape=jax.ShapeDtypeStruct((2 * c, h, w), jnp.float32),
        compiler_params=pltpu.CompilerParams(
            dimension_semantics=("arbitrary",),
        ),
    )(row_embed[:h], col_embed[:w])
    return out
